# baseline (speedup 1.0000x reference)
import functools
import math

import jax
import jax.numpy as jnp
from jax import lax
from jax.experimental import pallas as pl
from jax.experimental.pallas import tpu as pltpu
from jax.experimental.pallas import tpu_sc as plsc

N = 10000
E = 320000
D = 128

NC = 2
NS = 16
NW = NC * NS
EPW = E // NW
CH = 80
NCHUNK = EPW // CH
NBUF = 4
IBUF = 8
DCH = 128
EPP = 10240
DNCHUNK = EPP // DCH
NPAD = N + NS
RPT = 624
RPT_LAST = N - RPT * (NS - 1)
DEGW = 16

_SC_MESH = dict(core_axis_name="c", subcore_axis_name="s",
                num_cores=NC, num_subcores=NS)


def _rows_partition(s, fn):
    @pl.when(s < NS - 1)
    def _():
        fn(pl.multiple_of(s * RPT, 8), RPT)

    @pl.when(s == NS - 1)
    def _():
        fn((NS - 1) * RPT, RPT_LAST)


def _sc_degree(dstr, deg_init):
    mesh = plsc.VectorSubcoreMesh(**_SC_MESH)

    @functools.partial(
        pl.kernel,
        out_type=jax.ShapeDtypeStruct((NC, N, DEGW), jnp.float32),
        mesh=mesh,
        scratch_types=[
            pltpu.VMEM((DNCHUNK, DCH), jnp.int32),
            pltpu.VMEM((DCH, DEGW), jnp.float32),
            pltpu.VMEM_SHARED((NPAD, DEGW), jnp.float32),
        ],
    )
    def deg_kernel(dstr_hbm, init_hbm, out_hbm, dst_all, ones_v, acc_sh):
        c = lax.axis_index("c")
        s = lax.axis_index("s")
        wid = s * NC + c

        pltpu.sync_copy(dstr_hbm.at[wid], dst_all)

        def fill(r, carry):
            ones_v[r, :] = jnp.full((DEGW,), 1.0, jnp.float32)
            return carry
        lax.fori_loop(0, DCH, fill, 0)

        _rows_partition(s, lambda r0, n: pltpu.sync_copy(
            init_hbm.at[c, pl.ds(r0, n)], acc_sh.at[pl.ds(r0, n)]))
        plsc.subcore_barrier()

        def body(i, carry):
            pltpu.sync_copy(ones_v, acc_sh.at[dst_all.at[i]], add=True)
            return carry
        lax.fori_loop(0, DNCHUNK, body, 0)

        plsc.subcore_barrier()
        _rows_partition(s, lambda r0, n: pltpu.sync_copy(
            acc_sh.at[pl.ds(r0, n)], out_hbm.at[c, pl.ds(r0, n)]))

    return deg_kernel(dstr, deg_init)


def _sc_scatter(g, src, dst, zeros):
    mesh = plsc.VectorSubcoreMesh(**_SC_MESH)

    @functools.partial(
        pl.kernel,
        out_type=jax.ShapeDtypeStruct((NC, N, D), jnp.float32),
        mesh=mesh,
        scratch_types=(
            [pltpu.VMEM((1, CH), jnp.int32) for _ in range(IBUF)]
            + [pltpu.VMEM((1, CH), jnp.int32) for _ in range(IBUF)]
            + [pltpu.VMEM((CH, D), jnp.float32) for _ in range(NBUF)]
            + [pltpu.SemaphoreType.DMA for _ in range(2 * NBUF)]
            + [pltpu.SemaphoreType.DMA for _ in range(2 * IBUF)]
            + [pltpu.VMEM_SHARED((N, D), jnp.float32)]
        ),
    )
    def scatter_kernel(g_hbm, src_hbm, dst_hbm, z_hbm, out_hbm, *scr):
        si = scr[0:IBUF]
        di = scr[IBUF:2 * IBUF]
        o = 2 * IBUF
        rows = scr[o:o + NBUF]
        gsem = scr[o + NBUF:o + 2 * NBUF]
        ssem = scr[o + 2 * NBUF:o + 3 * NBUF]
        o2 = o + 3 * NBUF
        isem = scr[o2:o2 + IBUF]
        dsem = scr[o2 + IBUF:o2 + 2 * IBUF]
        acc_sh = scr[o2 + 2 * IBUF]
        c = lax.axis_index("c")
        s = lax.axis_index("s")
        wid = s * NC + c
        ebase = wid * EPW

        def fetch_idx(j):
            b = j % IBUF
            pltpu.async_copy(src_hbm.at[pl.ds(ebase + j * CH, CH)],
                             si[b].at[0], isem[b])
            pltpu.async_copy(dst_hbm.at[pl.ds(ebase + j * CH, CH)],
                             di[b].at[0], dsem[b])

        def issue_gather(j):
            b = j % NBUF
            bi = j % IBUF
            pltpu.make_async_copy(src_hbm.at[pl.ds(ebase + j * CH, CH)],
                                  si[bi].at[0], isem[bi]).wait()
            pltpu.async_copy(g_hbm.at[si[bi].at[0]], rows[b], gsem[b])

        @pl.when(c == 0)
        def _():
            _rows_partition(s, lambda r0, n: pltpu.sync_copy(
                g_hbm.at[pl.ds(r0, n)], acc_sh.at[pl.ds(r0, n)]))

        @pl.when(c != 0)
        def _():
            _rows_partition(s, lambda r0, n: pltpu.sync_copy(
                z_hbm.at[pl.ds(r0, n)], acc_sh.at[pl.ds(r0, n)]))

        plsc.subcore_barrier()

        for j in range(5):
            fetch_idx(j)
        issue_gather(0)
        issue_gather(1)

        def fetch_idx_at(j, ph):
            pltpu.async_copy(src_hbm.at[pl.ds(ebase + j * CH, CH)],
                             si[ph].at[0], isem[ph])
            pltpu.async_copy(dst_hbm.at[pl.ds(ebase + j * CH, CH)],
                             di[ph].at[0], dsem[ph])

        def gather_at(j, ph):
            pltpu.make_async_copy(src_hbm.at[pl.ds(ebase + j * CH, CH)],
                                  si[ph].at[0], isem[ph]).wait()
            pltpu.async_copy(g_hbm.at[si[ph].at[0]], rows[ph % NBUF],
                             gsem[ph % NBUF])

        def step(i, ph, w_sc, do_g, do_i):
            b = ph % NBUF
            b2 = (ph + 2) % NBUF
            if w_sc:
                pltpu.make_async_copy(
                    rows[b2], acc_sh.at[di[(ph - 2) % IBUF].at[0]],
                    ssem[b2]).wait()
            if do_g:
                gather_at(i + 2, (ph + 2) % IBUF)
            if do_i:
                fetch_idx_at(i + 5, (ph + 5) % IBUF)
            pltpu.make_async_copy(g_hbm.at[si[ph].at[0]], rows[b],
                                  gsem[b]).wait()
            pltpu.make_async_copy(dst_hbm.at[pl.ds(ebase + i * CH, CH)],
                                  di[ph].at[0], dsem[ph]).wait()
            pltpu.async_copy(rows[b], acc_sh.at[di[ph].at[0]], ssem[b],
                             add=True)

        for i in range(IBUF):
            step(i, i, i >= 2, i + 2 < NCHUNK, i + 5 < NCHUNK)
        nfull = (NCHUNK - IBUF - 5) // IBUF

        def octet(j, carry):
            for k in range(IBUF):
                step(IBUF * (j + 1) + k, k, True, True, True)
            return carry

        lax.fori_loop(0, nfull, octet, 0)
        for i in range(IBUF * (nfull + 1), NCHUNK):
            step(i, i % IBUF, True, i + 2 < NCHUNK, i + 5 < NCHUNK)

        for i in (NCHUNK - 2, NCHUNK - 1):
            b = i % NBUF
            pltpu.make_async_copy(rows[b], acc_sh.at[di[i % IBUF].at[0]],
                                  ssem[b]).wait()

        plsc.subcore_barrier()
        _rows_partition(s, lambda r0, n: pltpu.sync_copy(
            acc_sh.at[pl.ds(r0, n)], out_hbm.at[c, pl.ds(r0, n)]))

    return scatter_kernel(g, src, dst, zeros)


_RB = 1000
_GRID = N // _RB


def _tc0_body(x_ref, w_ref, h_ref):
    h_ref[...] = jnp.dot(x_ref[...], w_ref[...],
                         preferred_element_type=jnp.float32)


def _tc0(x, W1):
    return pl.pallas_call(
        _tc0_body,
        grid=(_GRID,),
        in_specs=[
            pl.BlockSpec((_RB, D), lambda r: (r, 0)),
            pl.BlockSpec((D, D), lambda r: (0, 0)),
        ],
        out_specs=pl.BlockSpec((_RB, D), lambda r: (r, 0)),
        out_shape=jax.ShapeDtypeStruct((N, D), jnp.float32),
    )(x, W1)


def _tc1_body(h_ref, deg_ref, g_ref, dinv_ref):
    deg = deg_ref[0, :, 0:1] + deg_ref[1, :, 0:1]
    dinv = lax.rsqrt(deg)
    g_ref[...] = h_ref[...] * dinv
    dinv_ref[...] = jnp.broadcast_to(dinv, (_RB, DEGW))


def _tc1(h, deg2):
    return pl.pallas_call(
        _tc1_body,
        grid=(_GRID,),
        in_specs=[
            pl.BlockSpec((_RB, D), lambda r: (r, 0)),
            pl.BlockSpec((NC, _RB, DEGW), lambda r: (0, r, 0)),
        ],
        out_specs=[
            pl.BlockSpec((_RB, D), lambda r: (r, 0)),
            pl.BlockSpec((_RB, DEGW), lambda r: (r, 0)),
        ],
        out_shape=[
            jax.ShapeDtypeStruct((N, D), jnp.float32),
            jax.ShapeDtypeStruct((N, DEGW), jnp.float32),
        ],
    )(h, deg2)


def _tc2_body(agg_ref, dinv_ref, b1_ref, w2_ref, g2_ref):
    a = agg_ref[0] + agg_ref[1]
    dinv = dinv_ref[:, 0:1]
    z = a * dinv + b1_ref[...]
    h = z * 0.5 * (1.0 + lax.erf(z * (1.0 / math.sqrt(2.0))))
    h2 = jnp.dot(h, w2_ref[...], preferred_element_type=jnp.float32)
    g2_ref[...] = h2 * dinv


def _tc2(agg1, dinv, b1, W2):
    return pl.pallas_call(
        _tc2_body,
        grid=(_GRID,),
        in_specs=[
            pl.BlockSpec((NC, _RB, D), lambda r: (0, r, 0)),
            pl.BlockSpec((_RB, DEGW), lambda r: (r, 0)),
            pl.BlockSpec((1, D), lambda r: (0, 0)),
            pl.BlockSpec((D, D), lambda r: (0, 0)),
        ],
        out_specs=pl.BlockSpec((_RB, D), lambda r: (r, 0)),
        out_shape=jax.ShapeDtypeStruct((N, D), jnp.float32),
    )(agg1, dinv, b1, W2)


def _tc3_body(agg_ref, dinv_ref, b2_ref, out_ref):
    a = agg_ref[0] + agg_ref[1]
    out_ref[...] = a * dinv_ref[:, 0:1] + b2_ref[...]


def _tc3(agg2, dinv, b2):
    return pl.pallas_call(
        _tc3_body,
        grid=(_GRID,),
        in_specs=[
            pl.BlockSpec((NC, _RB, D), lambda r: (0, r, 0)),
            pl.BlockSpec((_RB, DEGW), lambda r: (r, 0)),
            pl.BlockSpec((1, D), lambda r: (0, 0)),
        ],
        out_specs=pl.BlockSpec((_RB, D), lambda r: (r, 0)),
        out_shape=jax.ShapeDtypeStruct((N, D), jnp.float32),
    )(agg2, dinv, b2)


@jax.jit
def _run(x, edge_index, target, W1, b1, W2, b2):
    src = edge_index[0]
    dst = edge_index[1]
    dst2 = dst.reshape(NW, EPW)
    pad = jnp.zeros((NW, EPP - EPW), jnp.int32)
    pad_dst = pad + N + (jnp.arange(NW, dtype=jnp.int32) // NC)[:, None]
    dstr = jnp.concatenate([dst2, pad_dst], axis=1).reshape(
        NW, DNCHUNK, DCH)
    zeros = jnp.zeros((N, D), jnp.float32)
    deg_init = jnp.stack([jnp.ones((N, DEGW), jnp.float32),
                          jnp.zeros((N, DEGW), jnp.float32)])
    b1r = b1.reshape(1, D)
    b2r = b2.reshape(1, D)

    h1 = _tc0(x, W1)
    deg2 = _sc_degree(dstr, deg_init)
    g1, dinv = _tc1(h1, deg2)
    agg1 = _sc_scatter(g1, src, dst, zeros)
    g2 = _tc2(agg1, dinv, b1r, W2)
    agg2 = _sc_scatter(g2, src, dst, zeros)
    out = _tc3(agg2, dinv, b2r)
    return (out, target)


def kernel(x, edge_index, target, W1, b1, W2, b2):
    return _run(x, edge_index, target, W1, b1, W2, b2)


# speedup vs baseline: 1.0114x; 1.0114x over previous
import functools
import math

import jax
import jax.numpy as jnp
from jax import lax
from jax.experimental import pallas as pl
from jax.experimental.pallas import tpu as pltpu
from jax.experimental.pallas import tpu_sc as plsc

N = 10000
E = 320000
D = 128

NC = 2
NS = 16
NW = NC * NS
EPW = E // NW
CH = 80
NCHUNK = EPW // CH
NBUF = 4
IBUF = 8
DCH = 128
EPP = 10240
DNCHUNK = EPP // DCH
NPAD = N + NS
RPT = 624
RPT_LAST = N - RPT * (NS - 1)
DEGW = 16

_SC_MESH = dict(core_axis_name="c", subcore_axis_name="s",
                num_cores=NC, num_subcores=NS)


def _rows_partition(s, fn):
    @pl.when(s < NS - 1)
    def _():
        fn(pl.multiple_of(s * RPT, 8), RPT)

    @pl.when(s == NS - 1)
    def _():
        fn((NS - 1) * RPT, RPT_LAST)


def _sc_degree(dstr, deg_init):
    mesh = plsc.VectorSubcoreMesh(**_SC_MESH)

    @functools.partial(
        pl.kernel,
        out_type=jax.ShapeDtypeStruct((NC, N, DEGW), jnp.float32),
        mesh=mesh,
        scratch_types=[
            pltpu.VMEM((DNCHUNK, DCH), jnp.int32),
            pltpu.VMEM((DCH, DEGW), jnp.float32),
            pltpu.VMEM_SHARED((NPAD, DEGW), jnp.float32),
        ] + [pltpu.SemaphoreType.DMA for _ in range(4)],
    )
    def deg_kernel(dstr_hbm, init_hbm, out_hbm, dst_all, ones_v, acc_sh,
                   *dsem):
        c = lax.axis_index("c")
        s = lax.axis_index("s")
        wid = s * NC + c

        pltpu.sync_copy(dstr_hbm.at[wid], dst_all)

        def fill(r, carry):
            ones_v[r, :] = jnp.full((DEGW,), 1.0, jnp.float32)
            return carry
        lax.fori_loop(0, DCH, fill, 0)

        _rows_partition(s, lambda r0, n: pltpu.sync_copy(
            init_hbm.at[c, pl.ds(r0, n)], acc_sh.at[pl.ds(r0, n)]))
        plsc.subcore_barrier()

        def issue(i, k):
            pltpu.async_copy(ones_v, acc_sh.at[dst_all.at[i]], dsem[k],
                             add=True)

        def drain(i, k):
            pltpu.make_async_copy(ones_v, acc_sh.at[dst_all.at[i]],
                                  dsem[k]).wait()

        for k in range(4):
            issue(k, k)

        def group(j, carry):
            for k in range(4):
                i = 4 * j + k
                drain(i - 4, k)
                issue(i, k)
            return carry
        lax.fori_loop(1, DNCHUNK // 4, group, 0)
        for k in range(4):
            drain(DNCHUNK - 4 + k, k)

        plsc.subcore_barrier()
        _rows_partition(s, lambda r0, n: pltpu.sync_copy(
            acc_sh.at[pl.ds(r0, n)], out_hbm.at[c, pl.ds(r0, n)]))

    return deg_kernel(dstr, deg_init)


def _sc_scatter(g, src, dst, zeros):
    mesh = plsc.VectorSubcoreMesh(**_SC_MESH)

    @functools.partial(
        pl.kernel,
        out_type=jax.ShapeDtypeStruct((NC, N, D), jnp.float32),
        mesh=mesh,
        scratch_types=(
            [pltpu.VMEM((1, CH), jnp.int32) for _ in range(IBUF)]
            + [pltpu.VMEM((1, CH), jnp.int32) for _ in range(IBUF)]
            + [pltpu.VMEM((CH, D), jnp.float32) for _ in range(NBUF)]
            + [pltpu.SemaphoreType.DMA for _ in range(2 * NBUF)]
            + [pltpu.SemaphoreType.DMA for _ in range(2 * IBUF)]
            + [pltpu.VMEM_SHARED((N, D), jnp.float32)]
        ),
    )
    def scatter_kernel(g_hbm, src_hbm, dst_hbm, z_hbm, out_hbm, *scr):
        si = scr[0:IBUF]
        di = scr[IBUF:2 * IBUF]
        o = 2 * IBUF
        rows = scr[o:o + NBUF]
        gsem = scr[o + NBUF:o + 2 * NBUF]
        ssem = scr[o + 2 * NBUF:o + 3 * NBUF]
        o2 = o + 3 * NBUF
        isem = scr[o2:o2 + IBUF]
        dsem = scr[o2 + IBUF:o2 + 2 * IBUF]
        acc_sh = scr[o2 + 2 * IBUF]
        c = lax.axis_index("c")
        s = lax.axis_index("s")
        wid = s * NC + c
        ebase = wid * EPW

        def fetch_idx(j):
            b = j % IBUF
            pltpu.async_copy(src_hbm.at[pl.ds(ebase + j * CH, CH)],
                             si[b].at[0], isem[b])
            pltpu.async_copy(dst_hbm.at[pl.ds(ebase + j * CH, CH)],
                             di[b].at[0], dsem[b])

        def issue_gather(j):
            b = j % NBUF
            bi = j % IBUF
            pltpu.make_async_copy(src_hbm.at[pl.ds(ebase + j * CH, CH)],
                                  si[bi].at[0], isem[bi]).wait()
            pltpu.async_copy(g_hbm.at[si[bi].at[0]], rows[b], gsem[b])

        @pl.when(c == 0)
        def _():
            _rows_partition(s, lambda r0, n: pltpu.sync_copy(
                g_hbm.at[pl.ds(r0, n)], acc_sh.at[pl.ds(r0, n)]))

        @pl.when(c != 0)
        def _():
            _rows_partition(s, lambda r0, n: pltpu.sync_copy(
                z_hbm.at[pl.ds(r0, n)], acc_sh.at[pl.ds(r0, n)]))

        plsc.subcore_barrier()

        for j in range(5):
            fetch_idx(j)
        issue_gather(0)
        issue_gather(1)

        def fetch_idx_at(j, ph):
            pltpu.async_copy(src_hbm.at[pl.ds(ebase + j * CH, CH)],
                             si[ph].at[0], isem[ph])
            pltpu.async_copy(dst_hbm.at[pl.ds(ebase + j * CH, CH)],
                             di[ph].at[0], dsem[ph])

        def gather_at(j, ph):
            pltpu.make_async_copy(src_hbm.at[pl.ds(ebase + j * CH, CH)],
                                  si[ph].at[0], isem[ph]).wait()
            pltpu.async_copy(g_hbm.at[si[ph].at[0]], rows[ph % NBUF],
                             gsem[ph % NBUF])

        def step(i, ph, w_sc, do_g, do_i):
            b = ph % NBUF
            b2 = (ph + 2) % NBUF
            if w_sc:
                pltpu.make_async_copy(
                    rows[b2], acc_sh.at[di[(ph - 2) % IBUF].at[0]],
                    ssem[b2]).wait()
            if do_g:
                gather_at(i + 2, (ph + 2) % IBUF)
            if do_i:
                fetch_idx_at(i + 5, (ph + 5) % IBUF)
            pltpu.make_async_copy(g_hbm.at[si[ph].at[0]], rows[b],
                                  gsem[b]).wait()
            pltpu.make_async_copy(dst_hbm.at[pl.ds(ebase + i * CH, CH)],
                                  di[ph].at[0], dsem[ph]).wait()
            pltpu.async_copy(rows[b], acc_sh.at[di[ph].at[0]], ssem[b],
                             add=True)

        for i in range(IBUF):
            step(i, i, i >= 2, i + 2 < NCHUNK, i + 5 < NCHUNK)
        nfull = (NCHUNK - IBUF - 5) // IBUF

        def octet(j, carry):
            for k in range(IBUF):
                step(IBUF * (j + 1) + k, k, True, True, True)
            return carry

        lax.fori_loop(0, nfull, octet, 0)
        for i in range(IBUF * (nfull + 1), NCHUNK):
            step(i, i % IBUF, True, i + 2 < NCHUNK, i + 5 < NCHUNK)

        for i in (NCHUNK - 2, NCHUNK - 1):
            b = i % NBUF
            pltpu.make_async_copy(rows[b], acc_sh.at[di[i % IBUF].at[0]],
                                  ssem[b]).wait()

        plsc.subcore_barrier()
        _rows_partition(s, lambda r0, n: pltpu.sync_copy(
            acc_sh.at[pl.ds(r0, n)], out_hbm.at[c, pl.ds(r0, n)]))

    return scatter_kernel(g, src, dst, zeros)


_RB = 1000
_GRID = N // _RB


def _tc0_body(x_ref, w_ref, h_ref):
    h_ref[...] = jnp.dot(x_ref[...], w_ref[...],
                         preferred_element_type=jnp.float32)


def _tc0(x, W1):
    return pl.pallas_call(
        _tc0_body,
        grid=(_GRID,),
        in_specs=[
            pl.BlockSpec((_RB, D), lambda r: (r, 0)),
            pl.BlockSpec((D, D), lambda r: (0, 0)),
        ],
        out_specs=pl.BlockSpec((_RB, D), lambda r: (r, 0)),
        out_shape=jax.ShapeDtypeStruct((N, D), jnp.float32),
    )(x, W1)


def _tc1_body(h_ref, deg_ref, g_ref, dinv_ref):
    deg = deg_ref[0, :, 0:1] + deg_ref[1, :, 0:1]
    dinv = lax.rsqrt(deg)
    g_ref[...] = h_ref[...] * dinv
    dinv_ref[...] = jnp.broadcast_to(dinv, (_RB, DEGW))


def _tc1(h, deg2):
    return pl.pallas_call(
        _tc1_body,
        grid=(_GRID,),
        in_specs=[
            pl.BlockSpec((_RB, D), lambda r: (r, 0)),
            pl.BlockSpec((NC, _RB, DEGW), lambda r: (0, r, 0)),
        ],
        out_specs=[
            pl.BlockSpec((_RB, D), lambda r: (r, 0)),
            pl.BlockSpec((_RB, DEGW), lambda r: (r, 0)),
        ],
        out_shape=[
            jax.ShapeDtypeStruct((N, D), jnp.float32),
            jax.ShapeDtypeStruct((N, DEGW), jnp.float32),
        ],
    )(h, deg2)


def _tc2_body(agg_ref, dinv_ref, b1_ref, w2_ref, g2_ref):
    a = agg_ref[0] + agg_ref[1]
    dinv = dinv_ref[:, 0:1]
    z = a * dinv + b1_ref[...]
    h = z * 0.5 * (1.0 + lax.erf(z * (1.0 / math.sqrt(2.0))))
    h2 = jnp.dot(h, w2_ref[...], preferred_element_type=jnp.float32)
    g2_ref[...] = h2 * dinv


def _tc2(agg1, dinv, b1, W2):
    return pl.pallas_call(
        _tc2_body,
        grid=(_GRID,),
        in_specs=[
            pl.BlockSpec((NC, _RB, D), lambda r: (0, r, 0)),
            pl.BlockSpec((_RB, DEGW), lambda r: (r, 0)),
            pl.BlockSpec((1, D), lambda r: (0, 0)),
            pl.BlockSpec((D, D), lambda r: (0, 0)),
        ],
        out_specs=pl.BlockSpec((_RB, D), lambda r: (r, 0)),
        out_shape=jax.ShapeDtypeStruct((N, D), jnp.float32),
    )(agg1, dinv, b1, W2)


def _tc3_body(agg_ref, dinv_ref, b2_ref, out_ref):
    a = agg_ref[0] + agg_ref[1]
    out_ref[...] = a * dinv_ref[:, 0:1] + b2_ref[...]


def _tc3(agg2, dinv, b2):
    return pl.pallas_call(
        _tc3_body,
        grid=(_GRID,),
        in_specs=[
            pl.BlockSpec((NC, _RB, D), lambda r: (0, r, 0)),
            pl.BlockSpec((_RB, DEGW), lambda r: (r, 0)),
            pl.BlockSpec((1, D), lambda r: (0, 0)),
        ],
        out_specs=pl.BlockSpec((_RB, D), lambda r: (r, 0)),
        out_shape=jax.ShapeDtypeStruct((N, D), jnp.float32),
    )(agg2, dinv, b2)


@jax.jit
def _run(x, edge_index, target, W1, b1, W2, b2):
    src = edge_index[0]
    dst = edge_index[1]
    dst2 = dst.reshape(NW, EPW)
    pad = jnp.zeros((NW, EPP - EPW), jnp.int32)
    pad_dst = pad + N + (jnp.arange(NW, dtype=jnp.int32) // NC)[:, None]
    dstr = jnp.concatenate([dst2, pad_dst], axis=1).reshape(
        NW, DNCHUNK, DCH)
    zeros = jnp.zeros((N, D), jnp.float32)
    deg_init = jnp.stack([jnp.ones((N, DEGW), jnp.float32),
                          jnp.zeros((N, DEGW), jnp.float32)])
    b1r = b1.reshape(1, D)
    b2r = b2.reshape(1, D)

    h1 = _tc0(x, W1)
    deg2 = _sc_degree(dstr, deg_init)
    g1, dinv = _tc1(h1, deg2)
    agg1 = _sc_scatter(g1, src, dst, zeros)
    g2 = _tc2(agg1, dinv, b1r, W2)
    agg2 = _sc_scatter(g2, src, dst, zeros)
    out = _tc3(agg2, dinv, b2r)
    return (out, target)


def kernel(x, edge_index, target, W1, b1, W2, b2):
    return _run(x, edge_index, target, W1, b1, W2, b2)
